# baseline (speedup 1.0000x reference)
import jax
import jax.numpy as jnp
from jax import lax
from jax.experimental import pallas as pl
from jax.experimental.pallas import tpu as pltpu
from jax.experimental.pallas import tpu_sc as plsc

_N = 10000
_D = 128
_H = _D // 2
_E = 300000
_LANES = 16
_NTILES = 16
_K = 128
_U = 1
_NB = 148
_NP = _NB // _U
_EPT = _NB * _K
_R = 10240
_RPT = _R // _NTILES


def _matmul_body(x_ref, w_ref, xi_ref, o_ref, o0_ref, o1_ref, i0_ref, i1_ref):
    xu = jnp.dot(x_ref[...], w_ref[...], preferred_element_type=jnp.float32)
    o_ref[...] = xu
    o0_ref[...] = xu[:, :_H]
    o1_ref[...] = xu[:, _H:]
    xi = xi_ref[...]
    i0_ref[...] = xi[:, :_H]
    i1_ref[...] = xi[:, _H:]


def _project(x_user, w_user, x_item):
    return pl.pallas_call(
        _matmul_body,
        out_shape=[
            jax.ShapeDtypeStruct((_N, _D), jnp.float32),
            jax.ShapeDtypeStruct((_N, _H), jnp.float32),
            jax.ShapeDtypeStruct((_N, _H), jnp.float32),
            jax.ShapeDtypeStruct((_N, _H), jnp.float32),
            jax.ShapeDtypeStruct((_N, _H), jnp.float32),
        ],
    )(x_user, w_user, x_item)


def _sc_body(xu0, xu1, xi0, xi1, src_ui, dst_ui, src_iu, dst_iu,
             zrow, zdeg, ones_hbm,
             sum_ui0, sum_ui1, deg_ui, sum_iu0, sum_iu1, deg_iu,
             idx_s, idx_d, rows_v, ones_v, acc_sh, deg_sh, gsem, ssem, dsem):
    c = lax.axis_index("c")
    s = lax.axis_index("s")
    r0 = s * _RPT

    def run(tab0, tab1, src_hbm, dst_hbm, sum0_hbm, sum1_hbm, deg_hbm):
        pltpu.sync_copy(src_hbm.at[s], idx_s)
        pltpu.sync_copy(dst_hbm.at[s], idx_d)
        pltpu.sync_copy(ones_hbm, ones_v)
        pltpu.sync_copy(zrow.at[pl.ds(r0, _RPT)], acc_sh.at[pl.ds(r0, _RPT)])
        pltpu.sync_copy(zdeg.at[pl.ds(r0, _RPT)], deg_sh.at[pl.ds(r0, _RPT)])
        plsc.subcore_barrier()

        def pipeline(tab, with_deg):
            for u in range(_U):
                pltpu.async_copy(tab.at[idx_s.at[u]], rows_v.at[u], gsem)

            def phase(p, carry):
                g = p % 2
                base = g * _U
                j0 = p * _U
                for u in range(_U):
                    pltpu.make_async_copy(
                        tab.at[idx_s.at[j0 + u]],
                        rows_v.at[base + u], gsem).wait()
                @pl.when(p >= 1)
                def _():
                    for u in range(_U):
                        pltpu.make_async_copy(
                            rows_v.at[base + u],
                            acc_sh.at[idx_d.at[j0 + u]], ssem).wait()
                        if with_deg:
                            pltpu.make_async_copy(
                                ones_v, deg_sh.at[idx_d.at[j0 + u]],
                                dsem).wait()
                @pl.when(p + 1 < _NP)
                def _():
                    nbase = (1 - g) * _U
                    for u in range(_U):
                        pltpu.async_copy(
                            tab.at[idx_s.at[j0 + _U + u]],
                            rows_v.at[nbase + u], gsem)
                for u in range(_U):
                    pltpu.async_copy(
                        rows_v.at[base + u],
                        acc_sh.at[idx_d.at[j0 + u]], ssem, add=True)
                    if with_deg:
                        pltpu.async_copy(
                            ones_v, deg_sh.at[idx_d.at[j0 + u]],
                            dsem, add=True)
                return carry

            lax.fori_loop(0, _NP, phase, 0)
            for u in range(_U):
                pltpu.make_async_copy(
                    rows_v.at[u], acc_sh.at[idx_d.at[u]], ssem).wait()
                if with_deg:
                    pltpu.make_async_copy(
                        ones_v, deg_sh.at[idx_d.at[u]], dsem).wait()

        pipeline(tab0, True)
        plsc.subcore_barrier()
        pltpu.sync_copy(acc_sh.at[pl.ds(r0, _RPT)], sum0_hbm.at[pl.ds(r0, _RPT)])
        pltpu.sync_copy(deg_sh.at[pl.ds(r0, _RPT)], deg_hbm.at[pl.ds(r0, _RPT)])
        pltpu.sync_copy(zrow.at[pl.ds(r0, _RPT)], acc_sh.at[pl.ds(r0, _RPT)])
        plsc.subcore_barrier()

        pipeline(tab1, False)
        plsc.subcore_barrier()
        pltpu.sync_copy(acc_sh.at[pl.ds(r0, _RPT)], sum1_hbm.at[pl.ds(r0, _RPT)])

    @pl.when(c == 0)
    def _():
        run(xu0, xu1, src_ui, dst_ui, sum_ui0, sum_ui1, deg_ui)

    @pl.when(c == 1)
    def _():
        run(xi0, xi1, src_iu, dst_iu, sum_iu0, sum_iu1, deg_iu)


def _aggregate(xu0, xu1, xi0, xi1, src_ui, dst_ui, src_iu, dst_iu):
    zrow = jnp.zeros((_R, _H), jnp.float32)
    zdeg = jnp.zeros((_R, _LANES), jnp.float32)
    ones = jnp.ones((_K, _LANES), jnp.float32)
    mesh = plsc.VectorSubcoreMesh(core_axis_name="c", subcore_axis_name="s")
    f = pl.kernel(
        _sc_body,
        out_type=[
            jax.ShapeDtypeStruct((_R, _H), jnp.float32),
            jax.ShapeDtypeStruct((_R, _H), jnp.float32),
            jax.ShapeDtypeStruct((_R, _LANES), jnp.float32),
            jax.ShapeDtypeStruct((_R, _H), jnp.float32),
            jax.ShapeDtypeStruct((_R, _H), jnp.float32),
            jax.ShapeDtypeStruct((_R, _LANES), jnp.float32),
        ],
        mesh=mesh,
        compiler_params=pltpu.CompilerParams(use_tc_tiling_on_sc=False),
        scratch_types=[
            pltpu.VMEM((_NB, _K), jnp.int32),
            pltpu.VMEM((_NB, _K), jnp.int32),
            pltpu.VMEM((2 * _U, _K, _H), jnp.float32),
            pltpu.VMEM((_K, _LANES), jnp.float32),
            pltpu.VMEM_SHARED((_R, _H), jnp.float32),
            pltpu.VMEM_SHARED((_R, _LANES), jnp.float32),
            pltpu.SemaphoreType.DMA,
            pltpu.SemaphoreType.DMA,
            pltpu.SemaphoreType.DMA,
        ],
    )
    return f(xu0, xu1, xi0, xi1, src_ui, dst_ui, src_iu, dst_iu,
             zrow, zdeg, ones)


def _div_body(su0, su1, du, si0, si1, di,
              out_u, degout_u, out_i, degout_i):
    deg_u = jnp.maximum(du[:_N, :], 1.0)
    inv_u = 1.0 / deg_u[:, 0:1]
    out_u[:, :_H] = su0[:_N, :] * inv_u
    out_u[:, _H:] = su1[:_N, :] * inv_u
    degout_u[...] = deg_u[:, 0]
    deg_i = jnp.maximum(di[:_N, :], 1.0)
    inv_i = 1.0 / deg_i[:, 0:1]
    out_i[:, :_H] = si0[:_N, :] * inv_i
    out_i[:, _H:] = si1[:_N, :] * inv_i
    degout_i[...] = deg_i[:, 0]


def _normalize(su0, su1, du, si0, si1, di):
    return pl.pallas_call(
        _div_body,
        out_shape=[
            jax.ShapeDtypeStruct((_N, _D), jnp.float32),
            jax.ShapeDtypeStruct((_N,), jnp.float32),
            jax.ShapeDtypeStruct((_N, _D), jnp.float32),
            jax.ShapeDtypeStruct((_N,), jnp.float32),
        ],
    )(su0, su1, du, si0, si1, di)


def _prep_indices(edge_index):
    pad = _NTILES * _EPT - _E
    src = jnp.concatenate([edge_index[0], jnp.zeros((pad,), jnp.int32)])
    dst = jnp.concatenate([edge_index[1], jnp.full((pad,), _N, jnp.int32)])
    return (src.reshape(_NTILES, _NB, _K), dst.reshape(_NTILES, _NB, _K))


def kernel(x_user, x_item, edge_index_ui, edge_index_iu, W_user):
    xu, xu0, xu1, xi0, xi1 = _project(x_user, W_user, x_item)
    src_ui, dst_ui = _prep_indices(edge_index_ui)
    src_iu, dst_iu = _prep_indices(edge_index_iu)
    sum_ui0, sum_ui1, deg2_ui, sum_iu0, sum_iu1, deg2_iu = _aggregate(
        xu0, xu1, xi0, xi1, src_ui, dst_ui, src_iu, dst_iu)
    out_ui, deg_ui, out_iu, deg_iu = _normalize(
        sum_ui0, sum_ui1, deg2_ui, sum_iu0, sum_iu1, deg2_iu)
    return (out_ui, xu, deg_ui, out_iu, x_item, deg_iu)


# speedup vs baseline: 1.0001x; 1.0001x over previous
import jax
import jax.numpy as jnp
from jax import lax
from jax.experimental import pallas as pl
from jax.experimental.pallas import tpu as pltpu
from jax.experimental.pallas import tpu_sc as plsc

_N = 10000
_D = 128
_H = _D // 2
_E = 300000
_LANES = 16
_NTILES = 16
_K = 128
_U = 1
_NB = 148
_NP = _NB // _U
_EPT = _NB * _K
_R = 10240
_RPT = _R // _NTILES


def _matmul_body(x_ref, w_ref, xi_ref, o_ref, o0_ref, o1_ref, i0_ref, i1_ref):
    xu = jnp.dot(x_ref[...], w_ref[...], preferred_element_type=jnp.float32)
    o_ref[...] = xu
    o0_ref[...] = xu[:, :_H]
    o1_ref[...] = xu[:, _H:]
    xi = xi_ref[...]
    i0_ref[...] = xi[:, :_H]
    i1_ref[...] = xi[:, _H:]


def _project(x_user, w_user, x_item):
    return pl.pallas_call(
        _matmul_body,
        out_shape=[
            jax.ShapeDtypeStruct((_N, _D), jnp.float32),
            jax.ShapeDtypeStruct((_N, _H), jnp.float32),
            jax.ShapeDtypeStruct((_N, _H), jnp.float32),
            jax.ShapeDtypeStruct((_N, _H), jnp.float32),
            jax.ShapeDtypeStruct((_N, _H), jnp.float32),
        ],
    )(x_user, w_user, x_item)


def _sc_body(xu0, xu1, xi0, xi1, src_ui, dst_ui, src_iu, dst_iu,
             zrow, zdeg, ones_hbm,
             sum_ui0, sum_ui1, deg_ui, sum_iu0, sum_iu1, deg_iu,
             idx_s, idx_d, rows_v, ones_v, acc_sh, deg_sh, gsem, ssem, dsem):
    c = lax.axis_index("c")
    s = lax.axis_index("s")
    r0 = s * _RPT

    def run(tab0, tab1, src_hbm, dst_hbm, sum0_hbm, sum1_hbm, deg_hbm):
        pltpu.sync_copy(src_hbm.at[s], idx_s)
        pltpu.sync_copy(dst_hbm.at[s], idx_d)
        pltpu.sync_copy(ones_hbm, ones_v)
        pltpu.sync_copy(zrow.at[pl.ds(r0, _RPT)], acc_sh.at[pl.ds(r0, _RPT)])
        pltpu.sync_copy(zdeg.at[pl.ds(r0, _RPT)], deg_sh.at[pl.ds(r0, _RPT)])
        plsc.subcore_barrier()

        def pipeline(tab, with_deg):
            pltpu.async_copy(tab.at[idx_s.at[0]], rows_v.at[0], gsem)

            def step(j, carry):
                b = j % 2
                nb = 1 - b
                pltpu.make_async_copy(
                    tab.at[idx_s.at[j]], rows_v.at[b], gsem).wait()
                @pl.when(j >= 1)
                def _():
                    pltpu.make_async_copy(
                        rows_v.at[nb], acc_sh.at[idx_d.at[j]], ssem).wait()

                @pl.when(j + 1 < _NB)
                def _():
                    pltpu.async_copy(
                        tab.at[idx_s.at[j + 1]], rows_v.at[nb], gsem)

                pltpu.async_copy(
                    rows_v.at[b], acc_sh.at[idx_d.at[j]], ssem, add=True)
                if with_deg:
                    pltpu.sync_copy(ones_v, deg_sh.at[idx_d.at[j]], add=True)
                return carry

            lax.fori_loop(0, _NB, step, 0)
            pltpu.make_async_copy(
                rows_v.at[(_NB - 1) % 2],
                acc_sh.at[idx_d.at[_NB - 1]], ssem).wait()

        pipeline(tab0, True)
        plsc.subcore_barrier()
        pltpu.sync_copy(acc_sh.at[pl.ds(r0, _RPT)], sum0_hbm.at[pl.ds(r0, _RPT)])
        pltpu.sync_copy(deg_sh.at[pl.ds(r0, _RPT)], deg_hbm.at[pl.ds(r0, _RPT)])
        pltpu.sync_copy(zrow.at[pl.ds(r0, _RPT)], acc_sh.at[pl.ds(r0, _RPT)])
        plsc.subcore_barrier()

        pipeline(tab1, False)
        plsc.subcore_barrier()
        pltpu.sync_copy(acc_sh.at[pl.ds(r0, _RPT)], sum1_hbm.at[pl.ds(r0, _RPT)])

    @pl.when(c == 0)
    def _():
        run(xu0, xu1, src_ui, dst_ui, sum_ui0, sum_ui1, deg_ui)

    @pl.when(c == 1)
    def _():
        run(xi0, xi1, src_iu, dst_iu, sum_iu0, sum_iu1, deg_iu)


def _aggregate(xu0, xu1, xi0, xi1, src_ui, dst_ui, src_iu, dst_iu):
    zrow = jnp.zeros((_R, _H), jnp.float32)
    zdeg = jnp.zeros((_R, _LANES), jnp.float32)
    ones = jnp.ones((_K, _LANES), jnp.float32)
    mesh = plsc.VectorSubcoreMesh(core_axis_name="c", subcore_axis_name="s")
    f = pl.kernel(
        _sc_body,
        out_type=[
            jax.ShapeDtypeStruct((_R, _H), jnp.float32),
            jax.ShapeDtypeStruct((_R, _H), jnp.float32),
            jax.ShapeDtypeStruct((_R, _LANES), jnp.float32),
            jax.ShapeDtypeStruct((_R, _H), jnp.float32),
            jax.ShapeDtypeStruct((_R, _H), jnp.float32),
            jax.ShapeDtypeStruct((_R, _LANES), jnp.float32),
        ],
        mesh=mesh,
        compiler_params=pltpu.CompilerParams(use_tc_tiling_on_sc=False),
        scratch_types=[
            pltpu.VMEM((_NB, _K), jnp.int32),
            pltpu.VMEM((_NB, _K), jnp.int32),
            pltpu.VMEM((2 * _U, _K, _H), jnp.float32),
            pltpu.VMEM((_K, _LANES), jnp.float32),
            pltpu.VMEM_SHARED((_R, _H), jnp.float32),
            pltpu.VMEM_SHARED((_R, _LANES), jnp.float32),
            pltpu.SemaphoreType.DMA,
            pltpu.SemaphoreType.DMA,
            pltpu.SemaphoreType.DMA,
        ],
    )
    return f(xu0, xu1, xi0, xi1, src_ui, dst_ui, src_iu, dst_iu,
             zrow, zdeg, ones)


def _div_body(su0, su1, du, si0, si1, di,
              out_u, degout_u, out_i, degout_i):
    deg_u = jnp.maximum(du[:_N, :], 1.0)
    inv_u = 1.0 / deg_u[:, 0:1]
    out_u[:, :_H] = su0[:_N, :] * inv_u
    out_u[:, _H:] = su1[:_N, :] * inv_u
    degout_u[...] = deg_u[:, 0]
    deg_i = jnp.maximum(di[:_N, :], 1.0)
    inv_i = 1.0 / deg_i[:, 0:1]
    out_i[:, :_H] = si0[:_N, :] * inv_i
    out_i[:, _H:] = si1[:_N, :] * inv_i
    degout_i[...] = deg_i[:, 0]


def _normalize(su0, su1, du, si0, si1, di):
    return pl.pallas_call(
        _div_body,
        out_shape=[
            jax.ShapeDtypeStruct((_N, _D), jnp.float32),
            jax.ShapeDtypeStruct((_N,), jnp.float32),
            jax.ShapeDtypeStruct((_N, _D), jnp.float32),
            jax.ShapeDtypeStruct((_N,), jnp.float32),
        ],
    )(su0, su1, du, si0, si1, di)


def _prep_indices(edge_index):
    pad = _NTILES * _EPT - _E
    src = jnp.concatenate([edge_index[0], jnp.zeros((pad,), jnp.int32)])
    dst = jnp.concatenate([edge_index[1], jnp.full((pad,), _N, jnp.int32)])
    return (src.reshape(_NTILES, _NB, _K), dst.reshape(_NTILES, _NB, _K))


def kernel(x_user, x_item, edge_index_ui, edge_index_iu, W_user):
    xu, xu0, xu1, xi0, xi1 = _project(x_user, W_user, x_item)
    src_ui, dst_ui = _prep_indices(edge_index_ui)
    src_iu, dst_iu = _prep_indices(edge_index_iu)
    sum_ui0, sum_ui1, deg2_ui, sum_iu0, sum_iu1, deg2_iu = _aggregate(
        xu0, xu1, xi0, xi1, src_ui, dst_ui, src_iu, dst_iu)
    out_ui, deg_ui, out_iu, deg_iu = _normalize(
        sum_ui0, sum_ui1, deg2_ui, sum_iu0, sum_iu1, deg2_iu)
    return (out_ui, xu, deg_ui, out_iu, x_item, deg_iu)


# speedup vs baseline: 1.0269x; 1.0268x over previous
import jax
import jax.numpy as jnp
from jax import lax
from jax.experimental import pallas as pl
from jax.experimental.pallas import tpu as pltpu
from jax.experimental.pallas import tpu_sc as plsc

_N = 10000
_D = 128
_H = _D // 2
_E = 300000
_LANES = 16
_NTILES = 16
_K = 128
_U = 1
_NB = 148
_NP = _NB // _U
_EPT = _NB * _K
_R = 10240
_RPT = _R // _NTILES


def _matmul_body(x_ref, w_ref, o_ref):
    o_ref[...] = jnp.dot(x_ref[...], w_ref[...],
                         preferred_element_type=jnp.float32)


def _project(x_user, w_user):
    return pl.pallas_call(
        _matmul_body,
        out_shape=jax.ShapeDtypeStruct((_N, _D), jnp.float32),
    )(x_user, w_user)


def _sc_body(xu0, xu1, xi0, xi1, src_ui, dst_ui, src_iu, dst_iu,
             zrow, zdeg, ones_hbm,
             sum_ui0, sum_ui1, deg_ui, sum_iu0, sum_iu1, deg_iu,
             idx_s, idx_d, rows_v, ones_v, acc_sh, deg_sh, gsem, ssem, dsem):
    c = lax.axis_index("c")
    s = lax.axis_index("s")
    r0 = s * _RPT

    def run(tab0, tab1, src_hbm, dst_hbm, sum0_hbm, sum1_hbm, deg_hbm):
        pltpu.sync_copy(src_hbm.at[s], idx_s)
        pltpu.sync_copy(dst_hbm.at[s], idx_d)
        pltpu.sync_copy(ones_hbm, ones_v)
        pltpu.sync_copy(zrow.at[pl.ds(r0, _RPT)], acc_sh.at[pl.ds(r0, _RPT)])
        pltpu.sync_copy(zdeg.at[pl.ds(r0, _RPT)], deg_sh.at[pl.ds(r0, _RPT)])
        plsc.subcore_barrier()

        def pipeline(tab, with_deg):
            pltpu.async_copy(tab.at[idx_s.at[0]], rows_v.at[0], gsem)

            def step(j, carry):
                b = j % 2
                nb = 1 - b
                pltpu.make_async_copy(
                    tab.at[idx_s.at[j]], rows_v.at[b], gsem).wait()
                @pl.when(j >= 1)
                def _():
                    pltpu.make_async_copy(
                        rows_v.at[nb], acc_sh.at[idx_d.at[j]], ssem).wait()

                @pl.when(j + 1 < _NB)
                def _():
                    pltpu.async_copy(
                        tab.at[idx_s.at[j + 1]], rows_v.at[nb], gsem)

                pltpu.async_copy(
                    rows_v.at[b], acc_sh.at[idx_d.at[j]], ssem, add=True)
                if with_deg:
                    pltpu.sync_copy(ones_v, deg_sh.at[idx_d.at[j]], add=True)
                return carry

            lax.fori_loop(0, _NB, step, 0)
            pltpu.make_async_copy(
                rows_v.at[(_NB - 1) % 2],
                acc_sh.at[idx_d.at[_NB - 1]], ssem).wait()

        pipeline(tab0, True)
        plsc.subcore_barrier()
        pltpu.sync_copy(acc_sh.at[pl.ds(r0, _RPT)], sum0_hbm.at[pl.ds(r0, _RPT)])
        pltpu.sync_copy(deg_sh.at[pl.ds(r0, _RPT)], deg_hbm.at[pl.ds(r0, _RPT)])
        pltpu.sync_copy(zrow.at[pl.ds(r0, _RPT)], acc_sh.at[pl.ds(r0, _RPT)])
        plsc.subcore_barrier()

        pipeline(tab1, False)
        plsc.subcore_barrier()
        pltpu.sync_copy(acc_sh.at[pl.ds(r0, _RPT)], sum1_hbm.at[pl.ds(r0, _RPT)])

    @pl.when(c == 0)
    def _():
        run(xu0, xu1, src_ui, dst_ui, sum_ui0, sum_ui1, deg_ui)

    @pl.when(c == 1)
    def _():
        run(xi0, xi1, src_iu, dst_iu, sum_iu0, sum_iu1, deg_iu)


def _aggregate(xu0, xu1, xi0, xi1, src_ui, dst_ui, src_iu, dst_iu):
    zrow = jnp.zeros((_R, _H), jnp.float32)
    zdeg = jnp.zeros((_R, _LANES), jnp.float32)
    ones = jnp.ones((_K, _LANES), jnp.float32)
    mesh = plsc.VectorSubcoreMesh(core_axis_name="c", subcore_axis_name="s")
    f = pl.kernel(
        _sc_body,
        out_type=[
            jax.ShapeDtypeStruct((_R, _H), jnp.float32),
            jax.ShapeDtypeStruct((_R, _H), jnp.float32),
            jax.ShapeDtypeStruct((_R, _LANES), jnp.float32),
            jax.ShapeDtypeStruct((_R, _H), jnp.float32),
            jax.ShapeDtypeStruct((_R, _H), jnp.float32),
            jax.ShapeDtypeStruct((_R, _LANES), jnp.float32),
        ],
        mesh=mesh,
        compiler_params=pltpu.CompilerParams(use_tc_tiling_on_sc=False),
        scratch_types=[
            pltpu.VMEM((_NB, _K), jnp.int32),
            pltpu.VMEM((_NB, _K), jnp.int32),
            pltpu.VMEM((2 * _U, _K, _H), jnp.float32),
            pltpu.VMEM((_K, _LANES), jnp.float32),
            pltpu.VMEM_SHARED((_R, _H), jnp.float32),
            pltpu.VMEM_SHARED((_R, _LANES), jnp.float32),
            pltpu.SemaphoreType.DMA,
            pltpu.SemaphoreType.DMA,
            pltpu.SemaphoreType.DMA,
        ],
    )
    return f(xu0, xu1, xi0, xi1, src_ui, dst_ui, src_iu, dst_iu,
             zrow, zdeg, ones)


def _div_body(su0, su1, du, si0, si1, di,
              out_u, degout_u, out_i, degout_i):
    deg_u = jnp.maximum(du[:_N, :], 1.0)
    inv_u = 1.0 / deg_u[:, 0:1]
    out_u[:, :_H] = su0[:_N, :] * inv_u
    out_u[:, _H:] = su1[:_N, :] * inv_u
    degout_u[...] = deg_u[:, 0]
    deg_i = jnp.maximum(di[:_N, :], 1.0)
    inv_i = 1.0 / deg_i[:, 0:1]
    out_i[:, :_H] = si0[:_N, :] * inv_i
    out_i[:, _H:] = si1[:_N, :] * inv_i
    degout_i[...] = deg_i[:, 0]


def _normalize(su0, su1, du, si0, si1, di):
    return pl.pallas_call(
        _div_body,
        out_shape=[
            jax.ShapeDtypeStruct((_N, _D), jnp.float32),
            jax.ShapeDtypeStruct((_N,), jnp.float32),
            jax.ShapeDtypeStruct((_N, _D), jnp.float32),
            jax.ShapeDtypeStruct((_N,), jnp.float32),
        ],
    )(su0, su1, du, si0, si1, di)


def _prep_indices(edge_index):
    pad = _NTILES * _EPT - _E
    src = jnp.concatenate([edge_index[0], jnp.zeros((pad,), jnp.int32)])
    dst = jnp.concatenate([edge_index[1], jnp.full((pad,), _N, jnp.int32)])
    return (src.reshape(_NTILES, _NB, _K), dst.reshape(_NTILES, _NB, _K))


def kernel(x_user, x_item, edge_index_ui, edge_index_iu, W_user):
    xu = _project(x_user, W_user)
    xu0 = jnp.copy(xu[:, :_H])
    xu1 = jnp.copy(xu[:, _H:])
    xi0 = jnp.copy(x_item[:, :_H])
    xi1 = jnp.copy(x_item[:, _H:])
    src_ui, dst_ui = _prep_indices(edge_index_ui)
    src_iu, dst_iu = _prep_indices(edge_index_iu)
    sum_ui0, sum_ui1, deg2_ui, sum_iu0, sum_iu1, deg2_iu = _aggregate(
        xu0, xu1, xi0, xi1, src_ui, dst_ui, src_iu, dst_iu)
    out_ui, deg_ui, out_iu, deg_iu = _normalize(
        sum_ui0, sum_ui1, deg2_ui, sum_iu0, sum_iu1, deg2_iu)
    return (out_ui, xu, deg_ui, out_iu, x_item, deg_iu)


# speedup vs baseline: 1.1915x; 1.1603x over previous
import jax
import jax.numpy as jnp
from jax import lax
from jax.experimental import pallas as pl
from jax.experimental.pallas import tpu as pltpu
from jax.experimental.pallas import tpu_sc as plsc

_N = 10000
_D = 128
_H = _D // 2
_E = 300000
_LANES = 16
_NTILES = 16
_K = 128
_NB = 147
_EPT = _NB * _K
_R = 10240
_RPT = _R // _NTILES


def _matmul_body(x_ref, w_ref, o_ref):
    o_ref[...] = jnp.dot(x_ref[...], w_ref[...],
                         preferred_element_type=jnp.float32)


def _project(x_user, w_user):
    return pl.pallas_call(
        _matmul_body,
        out_shape=jax.ShapeDtypeStruct((_N, _D), jnp.float32),
    )(x_user, w_user)


def _sc_body(xu0, xu1, xi0, xi1, src_ui, dst_ui, src_iu, dst_iu,
             zrow, zdeg, ones_hbm,
             sum_ui0, sum_ui1, deg_ui, sum_iu0, sum_iu1, deg_iu,
             idx_s, idx_d, rows_v, ones_v, acc_sh, deg_sh, gsem, ssem):
    c = lax.axis_index("c")
    s = lax.axis_index("s")
    r0 = s * _RPT

    def run(tab0, tab1, src_hbm, dst_hbm, sum0_hbm, sum1_hbm, deg_hbm):
        pltpu.sync_copy(src_hbm.at[s], idx_s)
        pltpu.sync_copy(dst_hbm.at[s], idx_d)
        pltpu.sync_copy(ones_hbm, ones_v)
        pltpu.sync_copy(zrow.at[pl.ds(r0, _RPT)], acc_sh.at[pl.ds(r0, _RPT)])
        pltpu.sync_copy(zdeg.at[pl.ds(r0, _RPT)], deg_sh.at[pl.ds(r0, _RPT)])
        plsc.subcore_barrier()

        def pipeline(tab, with_deg):
            pltpu.async_copy(tab.at[idx_s.at[0]], rows_v.at[0], gsem)

            def step(j, carry):
                b = j % 2
                nb = 1 - b
                pltpu.make_async_copy(
                    tab.at[idx_s.at[j]], rows_v.at[b], gsem).wait()
                @pl.when(j >= 1)
                def _():
                    pltpu.make_async_copy(
                        rows_v.at[nb], acc_sh.at[idx_d.at[j]], ssem).wait()

                @pl.when(j + 1 < _NB)
                def _():
                    pltpu.async_copy(
                        tab.at[idx_s.at[j + 1]], rows_v.at[nb], gsem)

                pltpu.async_copy(
                    rows_v.at[b], acc_sh.at[idx_d.at[j]], ssem, add=True)
                if with_deg:
                    pltpu.sync_copy(ones_v, deg_sh.at[idx_d.at[j]], add=True)
                return carry

            lax.fori_loop(0, _NB, step, 0)
            pltpu.make_async_copy(
                rows_v.at[(_NB - 1) % 2],
                acc_sh.at[idx_d.at[_NB - 1]], ssem).wait()

        pipeline(tab0, True)
        plsc.subcore_barrier()
        pltpu.sync_copy(acc_sh.at[pl.ds(r0, _RPT)], sum0_hbm.at[pl.ds(r0, _RPT)])
        pltpu.sync_copy(deg_sh.at[pl.ds(r0, _RPT)], deg_hbm.at[pl.ds(r0, _RPT)])
        pltpu.sync_copy(zrow.at[pl.ds(r0, _RPT)], acc_sh.at[pl.ds(r0, _RPT)])
        plsc.subcore_barrier()

        pipeline(tab1, False)
        plsc.subcore_barrier()
        pltpu.sync_copy(acc_sh.at[pl.ds(r0, _RPT)], sum1_hbm.at[pl.ds(r0, _RPT)])

    @pl.when(c == 0)
    def _():
        run(xu0, xu1, src_ui, dst_ui, sum_ui0, sum_ui1, deg_ui)

    @pl.when(c == 1)
    def _():
        run(xi0, xi1, src_iu, dst_iu, sum_iu0, sum_iu1, deg_iu)


def _aggregate(xu0, xu1, xi0, xi1, src_ui, dst_ui, src_iu, dst_iu):
    zrow = jnp.zeros((_R, _H), jnp.float32)
    zdeg = jnp.zeros((_R, _LANES), jnp.float32)
    ones = jnp.ones((_K, _LANES), jnp.float32)
    mesh = plsc.VectorSubcoreMesh(core_axis_name="c", subcore_axis_name="s")
    f = pl.kernel(
        _sc_body,
        out_type=[
            jax.ShapeDtypeStruct((_R, _H), jnp.float32),
            jax.ShapeDtypeStruct((_R, _H), jnp.float32),
            jax.ShapeDtypeStruct((_R, _LANES), jnp.float32),
            jax.ShapeDtypeStruct((_R, _H), jnp.float32),
            jax.ShapeDtypeStruct((_R, _H), jnp.float32),
            jax.ShapeDtypeStruct((_R, _LANES), jnp.float32),
        ],
        mesh=mesh,
        compiler_params=pltpu.CompilerParams(use_tc_tiling_on_sc=False),
        scratch_types=[
            pltpu.VMEM((_NB, _K), jnp.int32),
            pltpu.VMEM((_NB, _K), jnp.int32),
            pltpu.VMEM((2, _K, _H), jnp.float32),
            pltpu.VMEM((_K, _LANES), jnp.float32),
            pltpu.VMEM_SHARED((_R, _H), jnp.float32),
            pltpu.VMEM_SHARED((_R, _LANES), jnp.float32),
            pltpu.SemaphoreType.DMA,
            pltpu.SemaphoreType.DMA,
        ],
    )
    return f(xu0, xu1, xi0, xi1, src_ui, dst_ui, src_iu, dst_iu,
             zrow, zdeg, ones)


def _div_body(sum0_ref, sum1_ref, deg_ref, out_ref, degout_ref):
    deg = jnp.maximum(deg_ref[...], 1.0)
    inv = 1.0 / deg[:, 0:1]
    out_ref[:, :_H] = sum0_ref[...] * inv
    out_ref[:, _H:] = sum1_ref[...] * inv
    degout_ref[...] = deg


_BLK = 1000


def _normalize(sum0, sum1, deg_r):
    return pl.pallas_call(
        _div_body,
        grid=(_N // _BLK,),
        in_specs=[
            pl.BlockSpec((_BLK, _H), lambda i: (i, 0)),
            pl.BlockSpec((_BLK, _H), lambda i: (i, 0)),
            pl.BlockSpec((_BLK, _LANES), lambda i: (i, 0)),
        ],
        out_specs=[
            pl.BlockSpec((_BLK, _D), lambda i: (i, 0)),
            pl.BlockSpec((_BLK, _LANES), lambda i: (i, 0)),
        ],
        out_shape=[
            jax.ShapeDtypeStruct((_N, _D), jnp.float32),
            jax.ShapeDtypeStruct((_N, _LANES), jnp.float32),
        ],
    )(sum0, sum1, deg_r)


def _prep_indices(edge_index):
    pad = _NTILES * _EPT - _E
    src = jnp.concatenate([edge_index[0], jnp.zeros((pad,), jnp.int32)])
    dst = jnp.concatenate([edge_index[1], jnp.full((pad,), _N, jnp.int32)])
    return (src.reshape(_NTILES, _NB, _K), dst.reshape(_NTILES, _NB, _K))


def kernel(x_user, x_item, edge_index_ui, edge_index_iu, W_user):
    xu = _project(x_user, W_user)
    xu0 = jnp.copy(xu[:, :_H])
    xu1 = jnp.copy(xu[:, _H:])
    xi0 = jnp.copy(x_item[:, :_H])
    xi1 = jnp.copy(x_item[:, _H:])
    src_ui, dst_ui = _prep_indices(edge_index_ui)
    src_iu, dst_iu = _prep_indices(edge_index_iu)
    sum_ui0, sum_ui1, deg2_ui, sum_iu0, sum_iu1, deg2_iu = _aggregate(
        xu0, xu1, xi0, xi1, src_ui, dst_ui, src_iu, dst_iu)
    out_ui, degc_ui = _normalize(sum_ui0, sum_ui1, deg2_ui)
    out_iu, degc_iu = _normalize(sum_iu0, sum_iu1, deg2_iu)
    return (out_ui, xu, degc_ui[:, 0], out_iu, x_item, degc_iu[:, 0])


# speedup vs baseline: 1.2025x; 1.0092x over previous
import jax
import jax.numpy as jnp
from jax import lax
from jax.experimental import pallas as pl
from jax.experimental.pallas import tpu as pltpu
from jax.experimental.pallas import tpu_sc as plsc

_N = 10000
_D = 128
_H = _D // 2
_E = 300000
_LANES = 16
_DEGW = 8
_NTILES = 16
_K = 128
_NB = 147
_EPT = _NB * _K
_R = 10240
_RPT = _R // _NTILES


def _matmul_body(x_ref, w_ref, o_ref):
    o_ref[...] = jnp.dot(x_ref[...], w_ref[...],
                         preferred_element_type=jnp.float32)


def _project(x_user, w_user):
    return pl.pallas_call(
        _matmul_body,
        out_shape=jax.ShapeDtypeStruct((_N, _D), jnp.float32),
    )(x_user, w_user)


def _sc_body(xu0, xu1, xi0, xi1, src_ui, dst_ui, src_iu, dst_iu,
             zrow, zdeg, ones_hbm,
             sum_ui0, sum_ui1, deg_ui, sum_iu0, sum_iu1, deg_iu,
             idx_s, idx_d, rows_v, ones_v, acc_sh, deg_sh, gsem, ssem):
    c = lax.axis_index("c")
    s = lax.axis_index("s")
    r0 = s * _RPT

    def run(tab0, tab1, src_hbm, dst_hbm, sum0_hbm, sum1_hbm, deg_hbm):
        pltpu.sync_copy(src_hbm.at[s], idx_s)
        pltpu.sync_copy(dst_hbm.at[s], idx_d)
        pltpu.sync_copy(ones_hbm, ones_v)
        pltpu.sync_copy(zrow.at[pl.ds(r0, _RPT)], acc_sh.at[pl.ds(r0, _RPT)])
        pltpu.sync_copy(zdeg.at[pl.ds(r0, _RPT)], deg_sh.at[pl.ds(r0, _RPT)])
        plsc.subcore_barrier()

        def pipeline(tab, with_deg):
            pltpu.async_copy(tab.at[idx_s.at[0]], rows_v.at[0], gsem)

            def step(j, carry):
                b = j % 2
                nb = 1 - b
                pltpu.make_async_copy(
                    tab.at[idx_s.at[j]], rows_v.at[b], gsem).wait()
                @pl.when(j >= 1)
                def _():
                    pltpu.make_async_copy(
                        rows_v.at[nb], acc_sh.at[idx_d.at[j]], ssem).wait()

                @pl.when(j + 1 < _NB)
                def _():
                    pltpu.async_copy(
                        tab.at[idx_s.at[j + 1]], rows_v.at[nb], gsem)

                pltpu.async_copy(
                    rows_v.at[b], acc_sh.at[idx_d.at[j]], ssem, add=True)
                if with_deg:
                    pltpu.sync_copy(ones_v, deg_sh.at[idx_d.at[j]], add=True)
                return carry

            lax.fori_loop(0, _NB, step, 0)
            pltpu.make_async_copy(
                rows_v.at[(_NB - 1) % 2],
                acc_sh.at[idx_d.at[_NB - 1]], ssem).wait()

        pipeline(tab0, True)
        plsc.subcore_barrier()
        pltpu.sync_copy(acc_sh.at[pl.ds(r0, _RPT)], sum0_hbm.at[pl.ds(r0, _RPT)])
        pltpu.sync_copy(deg_sh.at[pl.ds(r0, _RPT)], deg_hbm.at[pl.ds(r0, _RPT)])
        pltpu.sync_copy(zrow.at[pl.ds(r0, _RPT)], acc_sh.at[pl.ds(r0, _RPT)])
        plsc.subcore_barrier()

        pipeline(tab1, False)
        plsc.subcore_barrier()
        pltpu.sync_copy(acc_sh.at[pl.ds(r0, _RPT)], sum1_hbm.at[pl.ds(r0, _RPT)])

    @pl.when(c == 0)
    def _():
        run(xu0, xu1, src_ui, dst_ui, sum_ui0, sum_ui1, deg_ui)

    @pl.when(c == 1)
    def _():
        run(xi0, xi1, src_iu, dst_iu, sum_iu0, sum_iu1, deg_iu)


def _aggregate(xu0, xu1, xi0, xi1, src_ui, dst_ui, src_iu, dst_iu):
    zrow = jnp.zeros((_R, _H), jnp.float32)
    zdeg = jnp.zeros((_R, _DEGW), jnp.float32)
    ones = jnp.ones((_K, _DEGW), jnp.float32)
    mesh = plsc.VectorSubcoreMesh(core_axis_name="c", subcore_axis_name="s")
    f = pl.kernel(
        _sc_body,
        out_type=[
            jax.ShapeDtypeStruct((_R, _H), jnp.float32),
            jax.ShapeDtypeStruct((_R, _H), jnp.float32),
            jax.ShapeDtypeStruct((_R, _DEGW), jnp.float32),
            jax.ShapeDtypeStruct((_R, _H), jnp.float32),
            jax.ShapeDtypeStruct((_R, _H), jnp.float32),
            jax.ShapeDtypeStruct((_R, _DEGW), jnp.float32),
        ],
        mesh=mesh,
        compiler_params=pltpu.CompilerParams(use_tc_tiling_on_sc=False),
        scratch_types=[
            pltpu.VMEM((_NB, _K), jnp.int32),
            pltpu.VMEM((_NB, _K), jnp.int32),
            pltpu.VMEM((2, _K, _H), jnp.float32),
            pltpu.VMEM((_K, _DEGW), jnp.float32),
            pltpu.VMEM_SHARED((_R, _H), jnp.float32),
            pltpu.VMEM_SHARED((_R, _DEGW), jnp.float32),
            pltpu.SemaphoreType.DMA,
            pltpu.SemaphoreType.DMA,
        ],
    )
    return f(xu0, xu1, xi0, xi1, src_ui, dst_ui, src_iu, dst_iu,
             zrow, zdeg, ones)


def _div_body(sum0_ref, sum1_ref, deg_ref, out_ref, degout_ref):
    deg = jnp.maximum(deg_ref[...], 1.0)
    inv = 1.0 / deg[:, 0:1]
    out_ref[:, :_H] = sum0_ref[...] * inv
    out_ref[:, _H:] = sum1_ref[...] * inv
    degout_ref[...] = deg


_BLK = 1000


def _normalize(sum0, sum1, deg_r):
    return pl.pallas_call(
        _div_body,
        grid=(_N // _BLK,),
        in_specs=[
            pl.BlockSpec((_BLK, _H), lambda i: (i, 0)),
            pl.BlockSpec((_BLK, _H), lambda i: (i, 0)),
            pl.BlockSpec((_BLK, _DEGW), lambda i: (i, 0)),
        ],
        out_specs=[
            pl.BlockSpec((_BLK, _D), lambda i: (i, 0)),
            pl.BlockSpec((_BLK, _DEGW), lambda i: (i, 0)),
        ],
        out_shape=[
            jax.ShapeDtypeStruct((_N, _D), jnp.float32),
            jax.ShapeDtypeStruct((_N, _DEGW), jnp.float32),
        ],
    )(sum0, sum1, deg_r)


def _prep_indices(edge_index):
    pad = _NTILES * _EPT - _E
    src = jnp.concatenate([edge_index[0], jnp.zeros((pad,), jnp.int32)])
    dst = jnp.concatenate([edge_index[1], jnp.full((pad,), _N, jnp.int32)])
    return (src.reshape(_NTILES, _NB, _K), dst.reshape(_NTILES, _NB, _K))


def kernel(x_user, x_item, edge_index_ui, edge_index_iu, W_user):
    xu = _project(x_user, W_user)
    xu0 = jnp.copy(xu[:, :_H])
    xu1 = jnp.copy(xu[:, _H:])
    xi0 = jnp.copy(x_item[:, :_H])
    xi1 = jnp.copy(x_item[:, _H:])
    src_ui, dst_ui = _prep_indices(edge_index_ui)
    src_iu, dst_iu = _prep_indices(edge_index_iu)
    sum_ui0, sum_ui1, deg2_ui, sum_iu0, sum_iu1, deg2_iu = _aggregate(
        xu0, xu1, xi0, xi1, src_ui, dst_ui, src_iu, dst_iu)
    out_ui, degc_ui = _normalize(sum_ui0, sum_ui1, deg2_ui)
    out_iu, degc_iu = _normalize(sum_iu0, sum_iu1, deg2_iu)
    return (out_ui, xu, degc_ui[:, 0], out_iu, x_item, degc_iu[:, 0])
